# Initial kernel scaffold; baseline (speedup 1.0000x reference)
import jax
import jax.numpy as jnp
from jax.experimental import pallas as pl


def kernel(q, mu, Wij, dir_ij, idx_i, idx_j, n_atoms, W1, b1, W2, b2):
    raise NotImplementedError("write your pallas kernel here")


# speedup vs baseline: 8.0293x; 8.0293x over previous
import functools

import numpy as np
import jax
import jax.numpy as jnp
from jax import lax
from jax.experimental import pallas as pl
from jax.experimental.pallas import tpu as pltpu
from jax.experimental.pallas import tpu_sc as plsc

F = 128
FS = 32
NSHARD = 4
NTILES = 16
CCHUNK = 80


def _mlp_tc(q2, w1t, b1, w2pt, b2p):
    n = q2.shape[0]
    blk = 400

    def body(q_ref, w1t_ref, b1_ref, w2pt_ref, b2p_ref, out_ref):
        h = jnp.dot(q_ref[...], w1t_ref[...], preferred_element_type=jnp.float32)
        h = h + b1_ref[...]
        h = h * jax.nn.sigmoid(h)
        x = jnp.dot(h, w2pt_ref[...], preferred_element_type=jnp.float32)
        out_ref[...] = x + b2p_ref[...]

    return pl.pallas_call(
        body,
        grid=(n // blk,),
        in_specs=[
            pl.BlockSpec((blk, F), lambda i: (i, 0)),
            pl.BlockSpec((F, F), lambda i: (0, 0)),
            pl.BlockSpec((1, F), lambda i: (0, 0)),
            pl.BlockSpec((F, 3 * F), lambda i: (0, 0)),
            pl.BlockSpec((1, 3 * F), lambda i: (0, 0)),
        ],
        out_specs=pl.BlockSpec((blk, 3 * F), lambda i: (i, 0)),
        out_shape=jax.ShapeDtypeStruct((n, 3 * F), jnp.float32),
    )(q2, w1t, b1.reshape(1, F), w2pt, b2p.reshape(1, 3 * F))


def _edges_sc(x4, mu4, init4, wij, dirij, idx_i, idx_j):
    n = init4.shape[1]
    e = wij.shape[0]
    e_tile = e // NTILES
    n_tile = n // NTILES
    nchunk = e_tile // CCHUNK
    mesh = plsc.VectorSubcoreMesh(core_axis_name="c", subcore_axis_name="s")

    @functools.partial(
        pl.kernel,
        mesh=mesh,
        compiler_params=pltpu.CompilerParams(use_tc_tiling_on_sc=False),
        out_type=[
            jax.ShapeDtypeStruct((n, 1, F), jnp.float32),
            jax.ShapeDtypeStruct((n, 3, F), jnp.float32),
        ],
        scratch_types=[
            pltpu.VMEM_SHARED((n, F), jnp.float32),
            pltpu.VMEM((CCHUNK,), jnp.int32),
            pltpu.VMEM((CCHUNK,), jnp.int32),
            pltpu.VMEM((CCHUNK * 3 + 16,), jnp.float32),
            pltpu.VMEM((CCHUNK, FS), jnp.float32),
            pltpu.VMEM((CCHUNK, FS), jnp.float32),
            pltpu.VMEM((CCHUNK, FS), jnp.float32),
            pltpu.VMEM((CCHUNK, 3 * FS), jnp.float32),
            pltpu.VMEM((CCHUNK, 3 * FS), jnp.float32),
            pltpu.VMEM((CCHUNK, 4 * FS), jnp.float32),
            pltpu.SemaphoreType.DMA,
            pltpu.SemaphoreType.DMA,
        ],
    )
    def k(x4_h, mu4_h, init4_h, wij_h, dir_h, idxi_h, idxj_h, qout_h, muout_h,
          acc, idxi_v, idxj_v, dir_v, wd_v, wa_v, wb_v, xg_v, mug_v, pay_v,
          sem1, sem2):
        c = lax.axis_index("c")
        s = lax.axis_index("s")
        rows0 = s * n_tile
        ebase = s * e_tile

        for p in range(2):
            shard = 2 * p + c

            pltpu.sync_copy(init4_h.at[shard, pl.ds(rows0, n_tile)],
                            acc.at[pl.ds(rows0, n_tile)])
            plsc.subcore_barrier()

            def chunk_body(kk, _):
                e0 = ebase + kk * CCHUNK
                pltpu.sync_copy(idxi_h.at[pl.ds(e0, CCHUNK)], idxi_v)
                pltpu.sync_copy(idxj_h.at[pl.ds(e0, CCHUNK)], idxj_v)
                pltpu.sync_copy(dir_h.at[pl.ds(3 * e0, 3 * CCHUNK)],
                                dir_v.at[pl.ds(0, 3 * CCHUNK)])
                pltpu.sync_copy(
                    wij_h.at[pl.ds(e0, CCHUNK), pl.ds(shard * FS, FS)], wd_v)
                pltpu.sync_copy(
                    wij_h.at[pl.ds(e0, CCHUNK), pl.ds(F + shard * FS, FS)], wa_v)
                pltpu.sync_copy(
                    wij_h.at[pl.ds(e0, CCHUNK), pl.ds(2 * F + shard * FS, FS)], wb_v)
                for t in range(CCHUNK // 16):
                    idxj_v[pl.ds(16 * t, 16)] = idxj_v[pl.ds(16 * t, 16)] * 4 + shard
                g1 = pltpu.async_copy(x4_h.at[idxj_v], xg_v, sem1)
                g2 = pltpu.async_copy(mu4_h.at[idxj_v], mug_v, sem2)
                g1.wait()
                g2.wait()

                def edge_body(ei, _):
                    dvec = dir_v[pl.ds(3 * ei, 16)]
                    d0 = dvec[0]
                    d1 = dvec[1]
                    d2 = dvec[2]
                    for h in range(FS // 16):
                        hs = 16 * h
                        pay_v[ei, pl.ds(hs, 16)] = (
                            wd_v[ei, pl.ds(hs, 16)] * xg_v[ei, pl.ds(hs, 16)])
                        a = wa_v[ei, pl.ds(hs, 16)] * xg_v[ei, pl.ds(FS + hs, 16)]
                        b = wb_v[ei, pl.ds(hs, 16)] * xg_v[ei, pl.ds(2 * FS + hs, 16)]
                        for d, dd in enumerate((d0, d1, d2)):
                            pay_v[ei, pl.ds(FS + FS * d + hs, 16)] = (
                                a * dd + b * mug_v[ei, pl.ds(FS * d + hs, 16)])
                    return 0

                lax.fori_loop(0, CCHUNK, edge_body, 0)
                pltpu.sync_copy(pay_v, acc.at[idxi_v], add=True)
                return 0

            lax.fori_loop(0, nchunk, chunk_body, 0)
            plsc.subcore_barrier()

            col = shard * FS
            pltpu.sync_copy(acc.at[pl.ds(rows0, n_tile), pl.ds(0, FS)],
                            qout_h.at[pl.ds(rows0, n_tile), 0, pl.ds(col, FS)])
            for d in range(3):
                pltpu.sync_copy(
                    acc.at[pl.ds(rows0, n_tile), pl.ds(FS + FS * d, FS)],
                    muout_h.at[pl.ds(rows0, n_tile), d, pl.ds(col, FS)])
            if p == 0:
                plsc.subcore_barrier()

    return k(x4, mu4, init4, wij, dirij, idx_i, idx_j)


def kernel(q, mu, Wij, dir_ij, idx_i, idx_j, n_atoms, W1, b1, W2, b2):
    n = q.shape[0]
    e = Wij.shape[0]
    idx_i = idx_i.astype(jnp.int32)
    idx_j = idx_j.astype(jnp.int32)

    perm = np.concatenate([
        np.concatenate([np.arange(FS * s, FS * s + FS),
                        np.arange(F + FS * s, F + FS * s + FS),
                        np.arange(2 * F + FS * s, 2 * F + FS * s + FS)])
        for s in range(NSHARD)
    ])
    w2p = W2[perm]
    b2p = b2[perm]

    xp = _mlp_tc(q[:, 0, :], W1.T, b1, w2p.T, b2p)
    x4 = xp.reshape(n * NSHARD, 3 * FS)

    mu_r = mu.reshape(n, 3, NSHARD, FS).transpose(0, 2, 1, 3)
    mu4 = mu_r.reshape(n * NSHARD, 3 * FS)
    q_r = q.reshape(n, 1, NSHARD, FS).transpose(0, 2, 1, 3)
    init4 = jnp.concatenate([q_r, mu_r], axis=2)
    init4 = init4.transpose(1, 0, 2, 3).reshape(NSHARD, n, 4 * FS)

    q_out, mu_out = _edges_sc(x4, mu4, init4, Wij.reshape(e, 3 * F),
                              dir_ij.reshape(-1), idx_i, idx_j)
    return (q_out.astype(q.dtype), mu_out.astype(mu.dtype))


# speedup vs baseline: 10.9849x; 1.3681x over previous
import functools

import numpy as np
import jax
import jax.numpy as jnp
from jax import lax
from jax.experimental import pallas as pl
from jax.experimental.pallas import tpu as pltpu
from jax.experimental.pallas import tpu_sc as plsc

F = 128
FS = 32
NSHARD = 4
NTILES = 16
CCHUNK = 80


def _mlp_tc(q2, w1t, b1, w2pt, b2p):
    n = q2.shape[0]
    blk = 400

    def body(q_ref, w1t_ref, b1_ref, w2pt_ref, b2p_ref, out_ref):
        h = jnp.dot(q_ref[...], w1t_ref[...], preferred_element_type=jnp.float32)
        h = h + b1_ref[...]
        h = h * jax.nn.sigmoid(h)
        x = jnp.dot(h, w2pt_ref[...], preferred_element_type=jnp.float32)
        out_ref[...] = x + b2p_ref[...]

    return pl.pallas_call(
        body,
        grid=(n // blk,),
        in_specs=[
            pl.BlockSpec((blk, F), lambda i: (i, 0)),
            pl.BlockSpec((F, F), lambda i: (0, 0)),
            pl.BlockSpec((1, F), lambda i: (0, 0)),
            pl.BlockSpec((F, 3 * F), lambda i: (0, 0)),
            pl.BlockSpec((1, 3 * F), lambda i: (0, 0)),
        ],
        out_specs=pl.BlockSpec((blk, 3 * F), lambda i: (i, 0)),
        out_shape=jax.ShapeDtypeStruct((n, 3 * F), jnp.float32),
    )(q2, w1t, b1.reshape(1, F), w2pt, b2p.reshape(1, 3 * F))


def _edges_sc(x4, mu4, init4, wij, dirij, idx_i, idx_j):
    n = init4.shape[1]
    e = wij.shape[0]
    e_tile = e // NTILES
    n_tile = n // NTILES
    nchunk = e_tile // CCHUNK
    mesh = plsc.VectorSubcoreMesh(core_axis_name="c", subcore_axis_name="s")

    @functools.partial(
        pl.kernel,
        mesh=mesh,
        compiler_params=pltpu.CompilerParams(use_tc_tiling_on_sc=False),
        out_type=[
            jax.ShapeDtypeStruct((n, 1, F), jnp.float32),
            jax.ShapeDtypeStruct((n, 3, F), jnp.float32),
        ],
        scratch_types=[
            pltpu.VMEM_SHARED((n, F), jnp.float32),
            pltpu.VMEM((CCHUNK,), jnp.int32),
            pltpu.VMEM((CCHUNK,), jnp.int32),
            pltpu.VMEM((CCHUNK * 3 + 16,), jnp.float32),
            pltpu.VMEM((CCHUNK, FS), jnp.float32),
            pltpu.VMEM((CCHUNK, FS), jnp.float32),
            pltpu.VMEM((CCHUNK, FS), jnp.float32),
            pltpu.VMEM((CCHUNK, 3 * FS), jnp.float32),
            pltpu.VMEM((CCHUNK, 3 * FS), jnp.float32),
            pltpu.VMEM((CCHUNK, 4 * FS), jnp.float32),
            pltpu.SemaphoreType.DMA,
            pltpu.SemaphoreType.DMA,
            pltpu.SemaphoreType.DMA,
        ],
    )
    def k(x4_h, mu4_h, init4_h, wij_h, dir_h, idxi_h, idxj_h, qout_h, muout_h,
          acc, idxi_v, idxj_v, dir_v, wd_v, wa_v, wb_v, xg_v, mug_v, pay_v,
          sem1, sem2, sem3):
        c = lax.axis_index("c")
        s = lax.axis_index("s")
        rows0 = s * n_tile
        ebase = s * e_tile

        for p in range(2):
            shard = 2 * p + c

            pltpu.sync_copy(init4_h.at[shard, pl.ds(rows0, n_tile)],
                            acc.at[pl.ds(rows0, n_tile)])
            plsc.subcore_barrier()

            def chunk_body(kk, _):
                e0 = ebase + kk * CCHUNK
                hj = pltpu.async_copy(idxj_h.at[pl.ds(e0, CCHUNK)], idxj_v, sem1)
                lin = [
                    pltpu.async_copy(idxi_h.at[pl.ds(e0, CCHUNK)], idxi_v, sem2),
                    pltpu.async_copy(dir_h.at[pl.ds(3 * e0, 3 * CCHUNK)],
                                     dir_v.at[pl.ds(0, 3 * CCHUNK)], sem2),
                    pltpu.async_copy(
                        wij_h.at[pl.ds(e0, CCHUNK), pl.ds(shard * FS, FS)],
                        wd_v, sem2),
                    pltpu.async_copy(
                        wij_h.at[pl.ds(e0, CCHUNK), pl.ds(F + shard * FS, FS)],
                        wa_v, sem2),
                    pltpu.async_copy(
                        wij_h.at[pl.ds(e0, CCHUNK), pl.ds(2 * F + shard * FS, FS)],
                        wb_v, sem2),
                ]
                hj.wait()
                for t in range(CCHUNK // 16):
                    idxj_v[pl.ds(16 * t, 16)] = idxj_v[pl.ds(16 * t, 16)] * 4 + shard
                g1 = pltpu.async_copy(x4_h.at[idxj_v], xg_v, sem3)
                g2 = pltpu.async_copy(mu4_h.at[idxj_v], mug_v, sem3)
                for h in lin:
                    h.wait()
                g1.wait()
                g2.wait()

                def edge_body(ei, _):
                    dvec = dir_v[pl.ds(3 * ei, 16)]
                    d0 = dvec[0]
                    d1 = dvec[1]
                    d2 = dvec[2]
                    for h in range(FS // 16):
                        hs = 16 * h
                        pay_v[ei, pl.ds(hs, 16)] = (
                            wd_v[ei, pl.ds(hs, 16)] * xg_v[ei, pl.ds(hs, 16)])
                        a = wa_v[ei, pl.ds(hs, 16)] * xg_v[ei, pl.ds(FS + hs, 16)]
                        b = wb_v[ei, pl.ds(hs, 16)] * xg_v[ei, pl.ds(2 * FS + hs, 16)]
                        for d, dd in enumerate((d0, d1, d2)):
                            pay_v[ei, pl.ds(FS + FS * d + hs, 16)] = (
                                a * dd + b * mug_v[ei, pl.ds(FS * d + hs, 16)])
                    return 0

                lax.fori_loop(0, CCHUNK, edge_body, 0)
                pltpu.sync_copy(pay_v, acc.at[idxi_v], add=True)
                return 0

            lax.fori_loop(0, nchunk, chunk_body, 0)
            plsc.subcore_barrier()

            col = shard * FS
            pltpu.sync_copy(acc.at[pl.ds(rows0, n_tile), pl.ds(0, FS)],
                            qout_h.at[pl.ds(rows0, n_tile), 0, pl.ds(col, FS)])
            for d in range(3):
                pltpu.sync_copy(
                    acc.at[pl.ds(rows0, n_tile), pl.ds(FS + FS * d, FS)],
                    muout_h.at[pl.ds(rows0, n_tile), d, pl.ds(col, FS)])
            if p == 0:
                plsc.subcore_barrier()

    return k(x4, mu4, init4, wij, dirij, idx_i, idx_j)


def kernel(q, mu, Wij, dir_ij, idx_i, idx_j, n_atoms, W1, b1, W2, b2):
    n = q.shape[0]
    e = Wij.shape[0]
    idx_i = idx_i.astype(jnp.int32)
    idx_j = idx_j.astype(jnp.int32)

    perm = np.concatenate([
        np.concatenate([np.arange(FS * s, FS * s + FS),
                        np.arange(F + FS * s, F + FS * s + FS),
                        np.arange(2 * F + FS * s, 2 * F + FS * s + FS)])
        for s in range(NSHARD)
    ])
    w2p = W2[perm]
    b2p = b2[perm]

    xp = _mlp_tc(q[:, 0, :], W1.T, b1, w2p.T, b2p)
    x4 = xp.reshape(n * NSHARD, 3 * FS)

    mu_r = mu.reshape(n, 3, NSHARD, FS).transpose(0, 2, 1, 3)
    mu4 = mu_r.reshape(n * NSHARD, 3 * FS)
    q_r = q.reshape(n, 1, NSHARD, FS).transpose(0, 2, 1, 3)
    init4 = jnp.concatenate([q_r, mu_r], axis=2)
    init4 = init4.transpose(1, 0, 2, 3).reshape(NSHARD, n, 4 * FS)

    q_out, mu_out = _edges_sc(x4, mu4, init4, Wij.reshape(e, 3 * F),
                              dir_ij.reshape(-1), idx_i, idx_j)
    return (q_out.astype(q.dtype), mu_out.astype(mu.dtype))


# speedup vs baseline: 11.0900x; 1.0096x over previous
import functools

import numpy as np
import jax
import jax.numpy as jnp
from jax import lax
from jax.experimental import pallas as pl
from jax.experimental.pallas import tpu as pltpu
from jax.experimental.pallas import tpu_sc as plsc

F = 128
FS = 32
NSHARD = 4
NTILES = 16
CCHUNK = 80


def _mlp_tc(q2, w1t, b1, w2pt, b2p):
    n = q2.shape[0]
    blk = 400

    def body(q_ref, w1t_ref, b1_ref, w2pt_ref, b2p_ref, out_ref):
        h = jnp.dot(q_ref[...], w1t_ref[...], preferred_element_type=jnp.float32)
        h = h + b1_ref[...]
        h = h * jax.nn.sigmoid(h)
        x = jnp.dot(h, w2pt_ref[...], preferred_element_type=jnp.float32)
        out_ref[...] = x + b2p_ref[...]

    return pl.pallas_call(
        body,
        grid=(n // blk,),
        in_specs=[
            pl.BlockSpec((blk, F), lambda i: (i, 0)),
            pl.BlockSpec((F, F), lambda i: (0, 0)),
            pl.BlockSpec((1, F), lambda i: (0, 0)),
            pl.BlockSpec((F, 3 * F), lambda i: (0, 0)),
            pl.BlockSpec((1, 3 * F), lambda i: (0, 0)),
        ],
        out_specs=pl.BlockSpec((blk, 3 * F), lambda i: (i, 0)),
        out_shape=jax.ShapeDtypeStruct((n, 3 * F), jnp.float32),
    )(q2, w1t, b1.reshape(1, F), w2pt, b2p.reshape(1, 3 * F))


def _edges_sc(x4, mu4, init4, wij, dirij, idx_i, idx_j):
    n = init4.shape[1]
    e = wij.shape[0]
    e_tile = e // NTILES
    n_tile = n // NTILES
    nchunk = e_tile // CCHUNK
    mesh = plsc.VectorSubcoreMesh(core_axis_name="c", subcore_axis_name="s")

    @functools.partial(
        pl.kernel,
        mesh=mesh,
        compiler_params=pltpu.CompilerParams(use_tc_tiling_on_sc=False),
        out_type=[
            jax.ShapeDtypeStruct((n, 1, F), jnp.float32),
            jax.ShapeDtypeStruct((n, 3, F), jnp.float32),
        ],
        scratch_types=[
            pltpu.VMEM_SHARED((n, F), jnp.float32),
            pltpu.VMEM((CCHUNK,), jnp.int32),
            pltpu.VMEM((CCHUNK,), jnp.int32),
            pltpu.VMEM((CCHUNK * 3 + 16,), jnp.float32),
            pltpu.VMEM((CCHUNK, FS), jnp.float32),
            pltpu.VMEM((CCHUNK, FS), jnp.float32),
            pltpu.VMEM((CCHUNK, FS), jnp.float32),
            pltpu.VMEM((CCHUNK, 3 * FS), jnp.float32),
            pltpu.VMEM((CCHUNK, 3 * FS), jnp.float32),
            pltpu.VMEM((CCHUNK, 4 * FS), jnp.float32),
            pltpu.SemaphoreType.DMA,
            pltpu.SemaphoreType.DMA,
            pltpu.SemaphoreType.DMA,
        ],
    )
    def k(x4_h, mu4_h, init4_h, wij_h, dir_h, idxi_h, idxj_h, qout_h, muout_h,
          acc, idxi_v, idxj_v, dir_v, wd_v, wa_v, wb_v, xg_v, mug_v, pay_v,
          sem1, sem2, sem3):
        c = lax.axis_index("c")
        s = lax.axis_index("s")
        rows0 = s * n_tile
        ebase = s * e_tile

        for p in range(2):
            shard = 2 * p + c

            pltpu.sync_copy(init4_h.at[shard, pl.ds(rows0, n_tile)],
                            acc.at[pl.ds(rows0, n_tile)])
            plsc.subcore_barrier()

            def chunk_body(kk, _):
                e0 = ebase + kk * CCHUNK
                hj = pltpu.async_copy(idxj_h.at[pl.ds(e0, CCHUNK)], idxj_v, sem1)
                lin = [
                    pltpu.async_copy(idxi_h.at[pl.ds(e0, CCHUNK)], idxi_v, sem2),
                    pltpu.async_copy(dir_h.at[pl.ds(3 * e0, 3 * CCHUNK)],
                                     dir_v.at[pl.ds(0, 3 * CCHUNK)], sem2),
                    pltpu.async_copy(
                        wij_h.at[pl.ds(e0, CCHUNK), pl.ds(shard * FS, FS)],
                        wd_v, sem2),
                    pltpu.async_copy(
                        wij_h.at[pl.ds(e0, CCHUNK), pl.ds(F + shard * FS, FS)],
                        wa_v, sem2),
                    pltpu.async_copy(
                        wij_h.at[pl.ds(e0, CCHUNK), pl.ds(2 * F + shard * FS, FS)],
                        wb_v, sem2),
                ]
                hj.wait()
                g1 = pltpu.async_copy(x4_h.at[shard].at[idxj_v], xg_v, sem3)
                g2 = pltpu.async_copy(mu4_h.at[shard].at[idxj_v], mug_v, sem3)
                for h in lin:
                    h.wait()
                g1.wait()
                g2.wait()

                def edge_body(ei, _):
                    dvec = dir_v[pl.ds(3 * ei, 16)]
                    d0 = dvec[0]
                    d1 = dvec[1]
                    d2 = dvec[2]
                    for h in range(FS // 16):
                        hs = 16 * h
                        pay_v[ei, pl.ds(hs, 16)] = (
                            wd_v[ei, pl.ds(hs, 16)] * xg_v[ei, pl.ds(hs, 16)])
                        a = wa_v[ei, pl.ds(hs, 16)] * xg_v[ei, pl.ds(FS + hs, 16)]
                        b = wb_v[ei, pl.ds(hs, 16)] * xg_v[ei, pl.ds(2 * FS + hs, 16)]
                        for d, dd in enumerate((d0, d1, d2)):
                            pay_v[ei, pl.ds(FS + FS * d + hs, 16)] = (
                                a * dd + b * mug_v[ei, pl.ds(FS * d + hs, 16)])
                    return 0

                lax.fori_loop(0, CCHUNK, edge_body, 0, unroll=4)
                pltpu.sync_copy(pay_v, acc.at[idxi_v], add=True)
                return 0

            lax.fori_loop(0, nchunk, chunk_body, 0)
            plsc.subcore_barrier()

            col = shard * FS
            pltpu.sync_copy(acc.at[pl.ds(rows0, n_tile), pl.ds(0, FS)],
                            qout_h.at[pl.ds(rows0, n_tile), 0, pl.ds(col, FS)])
            for d in range(3):
                pltpu.sync_copy(
                    acc.at[pl.ds(rows0, n_tile), pl.ds(FS + FS * d, FS)],
                    muout_h.at[pl.ds(rows0, n_tile), d, pl.ds(col, FS)])
            if p == 0:
                plsc.subcore_barrier()

    return k(x4, mu4, init4, wij, dirij, idx_i, idx_j)


def kernel(q, mu, Wij, dir_ij, idx_i, idx_j, n_atoms, W1, b1, W2, b2):
    n = q.shape[0]
    e = Wij.shape[0]
    idx_i = idx_i.astype(jnp.int32)
    idx_j = idx_j.astype(jnp.int32)

    perm = np.concatenate([
        np.concatenate([np.arange(FS * s, FS * s + FS),
                        np.arange(F + FS * s, F + FS * s + FS),
                        np.arange(2 * F + FS * s, 2 * F + FS * s + FS)])
        for s in range(NSHARD)
    ])
    w2p = W2[perm]
    b2p = b2[perm]

    xp = _mlp_tc(q[:, 0, :], W1.T, b1, w2p.T, b2p)
    x4 = xp.reshape(n, NSHARD, 3 * FS).transpose(1, 0, 2)

    mu_r = mu.reshape(n, 3, NSHARD, FS).transpose(0, 2, 1, 3)
    mu4 = mu_r.reshape(n, NSHARD, 3 * FS).transpose(1, 0, 2)
    q_r = q.reshape(n, 1, NSHARD, FS).transpose(0, 2, 1, 3)
    init4 = jnp.concatenate([q_r, mu_r], axis=2)
    init4 = init4.transpose(1, 0, 2, 3).reshape(NSHARD, n, 4 * FS)

    q_out, mu_out = _edges_sc(x4, mu4, init4, Wij.reshape(e, 3 * F),
                              dir_ij.reshape(-1), idx_i, idx_j)
    return (q_out.astype(q.dtype), mu_out.astype(mu.dtype))


# speedup vs baseline: 12.2609x; 1.1056x over previous
import functools

import numpy as np
import jax
import jax.numpy as jnp
from jax import lax
from jax.experimental import pallas as pl
from jax.experimental.pallas import tpu as pltpu
from jax.experimental.pallas import tpu_sc as plsc

F = 128
FS = 32
NSHARD = 4
NTILES = 16
CCHUNK = 80


def _mlp_tc(q2, w1t, b1, w2pt, b2p):
    n = q2.shape[0]
    blk = 400

    def body(q_ref, w1t_ref, b1_ref, w2pt_ref, b2p_ref, out_ref):
        h = jnp.dot(q_ref[...], w1t_ref[...], preferred_element_type=jnp.float32)
        h = h + b1_ref[...]
        h = h * jax.nn.sigmoid(h)
        x = jnp.dot(h, w2pt_ref[...], preferred_element_type=jnp.float32)
        out_ref[...] = x + b2p_ref[...]

    return pl.pallas_call(
        body,
        grid=(n // blk,),
        in_specs=[
            pl.BlockSpec((blk, F), lambda i: (i, 0)),
            pl.BlockSpec((F, F), lambda i: (0, 0)),
            pl.BlockSpec((1, F), lambda i: (0, 0)),
            pl.BlockSpec((F, 3 * F), lambda i: (0, 0)),
            pl.BlockSpec((1, 3 * F), lambda i: (0, 0)),
        ],
        out_specs=pl.BlockSpec((blk, 3 * F), lambda i: (i, 0)),
        out_shape=jax.ShapeDtypeStruct((n, 3 * F), jnp.float32),
    )(q2, w1t, b1.reshape(1, F), w2pt, b2p.reshape(1, 3 * F))


def _edges_sc(x4, mu4, init4, wij, dirij, idx_i, idx_j):
    n = init4.shape[1]
    e = wij.shape[0]
    e_tile = e // NTILES
    n_tile = n // NTILES
    nchunk = e_tile // CCHUNK
    npair = (nchunk - 1) // 2
    mesh = plsc.VectorSubcoreMesh(core_axis_name="c", subcore_axis_name="s")
    vm = pltpu.VMEM

    @functools.partial(
        pl.kernel,
        mesh=mesh,
        compiler_params=pltpu.CompilerParams(use_tc_tiling_on_sc=False),
        out_type=[
            jax.ShapeDtypeStruct((n, 1, F), jnp.float32),
            jax.ShapeDtypeStruct((n, 3, F), jnp.float32),
        ],
        scratch_types=[
            pltpu.VMEM_SHARED((n, F), jnp.float32),
            [vm((CCHUNK,), jnp.int32)] * 2,
            [vm((CCHUNK,), jnp.int32)] * 2,
            [vm((CCHUNK * 3 + 16,), jnp.float32)] * 2,
            vm((CCHUNK, FS), jnp.float32),
            vm((CCHUNK, FS), jnp.float32),
            vm((CCHUNK, FS), jnp.float32),
            [vm((CCHUNK, 3 * FS), jnp.float32)] * 2,
            [vm((CCHUNK, 3 * FS), jnp.float32)] * 2,
            vm((CCHUNK, 4 * FS), jnp.float32),
            [pltpu.SemaphoreType.DMA] * 2,
            [pltpu.SemaphoreType.DMA] * 2,
            pltpu.SemaphoreType.DMA,
        ],
    )
    def k(x4_h, mu4_h, init4_h, wij_h, dir_h, idxi_h, idxj_h, qout_h, muout_h,
          acc, idxi_v, idxj_v, dir_v, wd_v, wa_v, wb_v, xg_v, mug_v, pay_v,
          semj, semlg, semw):
        c = lax.axis_index("c")
        s = lax.axis_index("s")
        rows0 = s * n_tile
        ebase = s * e_tile

        def issue_idx(kk, b):
            e0 = ebase + kk * CCHUNK
            pltpu.async_copy(idxj_h.at[pl.ds(e0, CCHUNK)], idxj_v[b], semj[b])
            pltpu.async_copy(idxi_h.at[pl.ds(e0, CCHUNK)], idxi_v[b], semlg[b])
            pltpu.async_copy(dir_h.at[pl.ds(3 * e0, 3 * CCHUNK)],
                             dir_v[b].at[pl.ds(0, 3 * CCHUNK)], semlg[b])

        def issue_wij(kk, shard):
            e0 = ebase + kk * CCHUNK
            for col, dst in ((0, wd_v), (F, wa_v), (2 * F, wb_v)):
                pltpu.async_copy(
                    wij_h.at[pl.ds(e0, CCHUNK), pl.ds(col + shard * FS, FS)],
                    dst, semw)

        def wait_j(b):
            pltpu.make_async_copy(idxj_h.at[pl.ds(0, CCHUNK)], idxj_v[b],
                                  semj[b]).wait()

        def wait_wij():
            for dst in (wd_v, wa_v, wb_v):
                pltpu.make_async_copy(
                    wij_h.at[pl.ds(0, CCHUNK), pl.ds(0, FS)], dst, semw).wait()

        def issue_gathers(b, shard):
            pltpu.async_copy(x4_h.at[shard].at[idxj_v[b]], xg_v[b], semlg[b])
            pltpu.async_copy(mu4_h.at[shard].at[idxj_v[b]], mug_v[b], semlg[b])

        def wait_lg(b):
            pltpu.make_async_copy(idxi_h.at[pl.ds(0, CCHUNK)], idxi_v[b],
                                  semlg[b]).wait()
            pltpu.make_async_copy(dir_h.at[pl.ds(0, 3 * CCHUNK)],
                                  dir_v[b].at[pl.ds(0, 3 * CCHUNK)],
                                  semlg[b]).wait()
            pltpu.make_async_copy(x4_h.at[0].at[idxj_v[b]], xg_v[b],
                                  semlg[b]).wait()
            pltpu.make_async_copy(mu4_h.at[0].at[idxj_v[b]], mug_v[b],
                                  semlg[b]).wait()

        def compute_scatter(b):
            def edge_body(ei, _):
                dvec = dir_v[b][pl.ds(3 * ei, 16)]
                d0 = dvec[0]
                d1 = dvec[1]
                d2 = dvec[2]
                for h in range(FS // 16):
                    hs = 16 * h
                    pay_v[ei, pl.ds(hs, 16)] = (
                        wd_v[ei, pl.ds(hs, 16)] * xg_v[b][ei, pl.ds(hs, 16)])
                    a = wa_v[ei, pl.ds(hs, 16)] * xg_v[b][ei, pl.ds(FS + hs, 16)]
                    bb = (wb_v[ei, pl.ds(hs, 16)]
                          * xg_v[b][ei, pl.ds(2 * FS + hs, 16)])
                    for d, dd in enumerate((d0, d1, d2)):
                        pay_v[ei, pl.ds(FS + FS * d + hs, 16)] = (
                            a * dd + bb * mug_v[b][ei, pl.ds(FS * d + hs, 16)])
                return 0

            lax.fori_loop(0, CCHUNK, edge_body, 0, unroll=4)
            pltpu.sync_copy(pay_v, acc.at[idxi_v[b]], add=True)

        for p in range(2):
            shard = 2 * p + c

            pltpu.sync_copy(init4_h.at[shard, pl.ds(rows0, n_tile)],
                            acc.at[pl.ds(rows0, n_tile)])
            plsc.subcore_barrier()

            issue_idx(0, 0)
            issue_idx(1, 1)
            issue_wij(0, shard)
            wait_j(0)
            issue_gathers(0, shard)

            def pair_body(kp, _):
                for b in (0, 1):
                    kk = 2 * kp + b
                    wait_j(1 - b)
                    issue_gathers(1 - b, shard)
                    wait_wij()
                    wait_lg(b)
                    compute_scatter(b)
                    issue_wij(kk + 1, shard)

                    @pl.when(kk + 2 < nchunk)
                    def _():
                        issue_idx(kk + 2, b)
                return 0

            lax.fori_loop(0, npair, pair_body, 0)
            wait_wij()
            wait_lg(0)
            compute_scatter(0)
            plsc.subcore_barrier()

            col = shard * FS
            pltpu.sync_copy(acc.at[pl.ds(rows0, n_tile), pl.ds(0, FS)],
                            qout_h.at[pl.ds(rows0, n_tile), 0, pl.ds(col, FS)])
            for d in range(3):
                pltpu.sync_copy(
                    acc.at[pl.ds(rows0, n_tile), pl.ds(FS + FS * d, FS)],
                    muout_h.at[pl.ds(rows0, n_tile), d, pl.ds(col, FS)])
            if p == 0:
                plsc.subcore_barrier()

    return k(x4, mu4, init4, wij, dirij, idx_i, idx_j)


def kernel(q, mu, Wij, dir_ij, idx_i, idx_j, n_atoms, W1, b1, W2, b2):
    n = q.shape[0]
    e = Wij.shape[0]
    idx_i = idx_i.astype(jnp.int32)
    idx_j = idx_j.astype(jnp.int32)

    perm = np.concatenate([
        np.concatenate([np.arange(FS * s, FS * s + FS),
                        np.arange(F + FS * s, F + FS * s + FS),
                        np.arange(2 * F + FS * s, 2 * F + FS * s + FS)])
        for s in range(NSHARD)
    ])
    w2p = W2[perm]
    b2p = b2[perm]

    xp = _mlp_tc(q[:, 0, :], W1.T, b1, w2p.T, b2p)
    x4 = xp.reshape(n, NSHARD, 3 * FS).transpose(1, 0, 2)

    mu_r = mu.reshape(n, 3, NSHARD, FS).transpose(0, 2, 1, 3)
    mu4 = mu_r.reshape(n, NSHARD, 3 * FS).transpose(1, 0, 2)
    q_r = q.reshape(n, 1, NSHARD, FS).transpose(0, 2, 1, 3)
    init4 = jnp.concatenate([q_r, mu_r], axis=2)
    init4 = init4.transpose(1, 0, 2, 3).reshape(NSHARD, n, 4 * FS)

    q_out, mu_out = _edges_sc(x4, mu4, init4, Wij.reshape(e, 3 * F),
                              dir_ij.reshape(-1), idx_i, idx_j)
    return (q_out.astype(q.dtype), mu_out.astype(mu.dtype))
